# baseline (speedup 1.0000x reference)
import functools

import jax
import jax.numpy as jnp
from jax import lax
from jax.experimental import pallas as pl
from jax.experimental.pallas import tpu as pltpu
from jax.experimental.pallas import tpu_sc as plsc


def _make_sc_copy(T, D, C=16):
    NW = 32
    rows_per_w = T // NW
    NCH = rows_per_w // C
    mesh = plsc.VectorSubcoreMesh(core_axis_name="c", subcore_axis_name="s")

    @functools.partial(
        pl.kernel,
        mesh=mesh,
        out_type=jax.ShapeDtypeStruct((T, D), jnp.float32),
        scratch_types=[
            pltpu.VMEM((2 * C, D), jnp.float32),
            pltpu.SemaphoreType.DMA,
            pltpu.SemaphoreType.DMA,
            pltpu.SemaphoreType.DMA,
            pltpu.SemaphoreType.DMA,
        ],
    )
    def copy_k(table_hbm, out_hbm, buf, sin0, sin1, sout0, sout1):
        wid = lax.axis_index("s") * 2 + lax.axis_index("c")
        base = wid * rows_per_w
        sin = (sin0, sin1)
        sout = (sout0, sout1)

        def in_dma(j):
            b = j % 2
            return pltpu.make_async_copy(
                table_hbm.at[pl.ds(base + j * C, C)],
                buf.at[pl.ds(b * C, C)],
                sin[b],
            )

        def out_dma(j):
            b = j % 2
            return pltpu.make_async_copy(
                buf.at[pl.ds(b * C, C)],
                out_hbm.at[pl.ds(base + j * C, C)],
                sout[b],
            )

        in_dma(0).start()
        for j in range(NCH):
            if j + 1 < NCH:
                if j >= 1:
                    out_dma(j - 1).wait()
                in_dma(j + 1).start()
            in_dma(j).wait()
            out_dma(j).start()
        out_dma(NCH - 2).wait()
        out_dma(NCH - 1).wait()

    return copy_k


def kernel(x, table):
    T = x.shape[1]
    D = table.shape[1]
    return _make_sc_copy(T, D)(table)


# speedup vs baseline: 1.5163x; 1.5163x over previous
import jax
import jax.numpy as jnp
from jax.experimental import pallas as pl
from jax.experimental.pallas import tpu as pltpu


def _copy_kernel(in_ref, out_ref):
    out_ref[...] = in_ref[...]


def _make_copy(T, D, BR):
    NB = T // BR
    return pl.pallas_call(
        _copy_kernel,
        grid=(NB,),
        in_specs=[pl.BlockSpec((BR, D), lambda i: (i, 0))],
        out_specs=pl.BlockSpec((BR, D), lambda i: (i, 0)),
        out_shape=jax.ShapeDtypeStruct((T, D), jnp.float32),
        compiler_params=pltpu.CompilerParams(
            dimension_semantics=("arbitrary",),
        ),
    )


def kernel(x, table):
    T = x.shape[1]
    D = table.shape[1]
    return _make_copy(T, D, 512)(table)
